# baseline (speedup 1.0000x reference)
import functools

import jax
import jax.numpy as jnp
from jax import lax
from jax.experimental import pallas as pl
from jax.experimental.pallas import tpu as pltpu
from jax.experimental.pallas import tpu_sc as plsc


def _gcn_kernel(a_ref, x_ref, w_ref, b_ref, z_ref):
    t = jnp.dot(a_ref[...], x_ref[...], preferred_element_type=jnp.float32)
    z_ref[...] = jnp.dot(t, w_ref[...],
                         preferred_element_type=jnp.float32) + b_ref[...]


_CHUNK = 120_000
_NCHUNK = 26
_PER_TILE = 3_125_000


def _sc_probe_kernel(a_flat, out_hbm, buf, sem):
    c = lax.axis_index("c")
    s = lax.axis_index("s")
    wid = s * 2 + c
    base = wid * _PER_TILE

    def body(j, carry):
        pltpu.sync_copy(a_flat.at[pl.ds(base + j * _CHUNK, _CHUNK)], buf)
        return carry

    lax.fori_loop(0, _NCHUNK, body, 0)
    pltpu.sync_copy(buf.at[pl.ds(0, 16)], out_hbm.at[pl.ds(wid * 16, 16)])


@jax.jit
def kernel(X, A_hat, W, b):
    n, d_in = X.shape
    d_out = W.shape[1]
    b2 = b.reshape(1, d_out)

    bm = 400
    grid = (pl.cdiv(n, bm),)
    Z = pl.pallas_call(
        _gcn_kernel,
        grid=grid,
        in_specs=[
            pl.BlockSpec((bm, n), lambda i: (i, 0)),
            pl.BlockSpec((n, d_in), lambda i: (0, 0)),
            pl.BlockSpec((d_in, d_out), lambda i: (0, 0)),
            pl.BlockSpec((1, d_out), lambda i: (0, 0)),
        ],
        out_specs=pl.BlockSpec((bm, d_out), lambda i: (i, 0)),
        out_shape=jax.ShapeDtypeStruct((n, d_out), jnp.float32),
    )(A_hat, X, W, b2)

    mesh = plsc.VectorSubcoreMesh(core_axis_name="c", subcore_axis_name="s")
    sc_probe = functools.partial(
        pl.kernel,
        mesh=mesh,
        out_type=jax.ShapeDtypeStruct((512,), jnp.float32),
        scratch_types=[
            pltpu.VMEM((_CHUNK,), jnp.float32),
            pltpu.SemaphoreType.DMA,
        ],
    )(_sc_probe_kernel)
    probe = sc_probe(A_hat.reshape(n * n))

    return Z.at[0, 0].add(probe[0] * 0.0)


# speedup vs baseline: 5.1785x; 5.1785x over previous
import jax
import jax.numpy as jnp
from jax.experimental import pallas as pl


def _gcn_kernel(a_ref, x_ref, w_ref, b_ref, z_ref):
    t = jnp.dot(a_ref[...], x_ref[...], preferred_element_type=jnp.float32)
    z_ref[...] = jnp.dot(t, w_ref[...],
                         preferred_element_type=jnp.float32) + b_ref[...]


@jax.jit
def kernel(X, A_hat, W, b):
    n, d_in = X.shape
    d_out = W.shape[1]
    b2 = b.reshape(1, d_out)

    bm = 400
    grid = (pl.cdiv(n, bm),)
    Z = pl.pallas_call(
        _gcn_kernel,
        grid=grid,
        in_specs=[
            pl.BlockSpec((bm, n), lambda i: (i, 0)),
            pl.BlockSpec((n, d_in), lambda i: (0, 0)),
            pl.BlockSpec((d_in, d_out), lambda i: (0, 0)),
            pl.BlockSpec((1, d_out), lambda i: (0, 0)),
        ],
        out_specs=pl.BlockSpec((bm, d_out), lambda i: (i, 0)),
        out_shape=jax.ShapeDtypeStruct((n, d_out), jnp.float32),
    )(A_hat, X, W, b2)
    return Z
